# baseline (speedup 1.0000x reference)
import functools

import jax
import jax.numpy as jnp
from jax import lax
from jax.experimental import pallas as pl
from jax.experimental.pallas import tpu as pltpu
from jax.experimental.pallas import tpu_sc as plsc

_NC = 2
_NS = 16
_NW = _NC * _NS

_GCHUNK = 256
_TBLK = 128


@functools.lru_cache(maxsize=None)
def _make_widen(V, D):
    V0 = (V // _TBLK) * _TBLK
    n_blocks = V0 // _TBLK
    per_tile = -(-n_blocks // _NW)
    mesh = plsc.VectorSubcoreMesh(core_axis_name="c", subcore_axis_name="s")

    @functools.partial(
        pl.kernel,
        out_type=jax.ShapeDtypeStruct((V, 128), jnp.float32),
        mesh=mesh,
        scratch_types=[
            pltpu.VMEM((D, _TBLK), jnp.float32),
            pltpu.VMEM((D, _TBLK), jnp.float32),
            pltpu.VMEM((_TBLK, 128), jnp.float32),
            pltpu.VMEM((_TBLK, 128), jnp.float32),
            pltpu.VMEM((D, 64), jnp.float32),
            pltpu.SemaphoreType.DMA,
            pltpu.SemaphoreType.DMA,
            pltpu.SemaphoreType.DMA,
            pltpu.SemaphoreType.DMA,
        ],
        compiler_params=pltpu.CompilerParams(
            use_tc_tiling_on_sc=True, needs_layout_passes=False),
    )
    def widen_kernel(wt_hbm, wtail_hbm, out_hbm, a0, a1, b0, b1, at_v,
                     si0, si1, so0, so1):
        a_v = (a0, a1)
        b_v = (b0, b1)
        sem_i = (si0, si1)
        sem_o = (so0, so1)
        wid = lax.axis_index("s") * _NC + lax.axis_index("c")
        lanes = lax.iota(jnp.int32, 16)

        def load_blk(c, rb):
            pltpu.async_copy(wt_hbm.at[:, pl.ds(c * _TBLK, _TBLK)],
                             a_v[rb], sem_i[rb])

        def wait_load(rb):
            pltpu.make_async_copy(wt_hbm.at[:, pl.ds(0, _TBLK)],
                                  a_v[rb], sem_i[rb]).wait()

        def transpose(rb):
            def rows(r8, carry2):
                for r in range(8):
                    row = r8 * 8 + r
                    cols = jax.lax.broadcast(row, (16,))
                    for k in range(D // 16):
                        b_v[rb][row, pl.ds(16 * k, 16)] = plsc.load_gather(
                            a_v[rb], [lanes + 16 * k, cols])
                return carry2
            lax.fori_loop(0, _TBLK // 8, rows, 0)

        def store_blk(c, rb):
            pltpu.async_copy(b_v[rb],
                             out_hbm.at[pl.ds(c * _TBLK, _TBLK), :],
                             sem_o[rb])

        def wait_store(rb):
            pltpu.make_async_copy(b_v[rb],
                                  out_hbm.at[pl.ds(0, _TBLK), :],
                                  sem_o[rb]).wait()

        def body(i, carry):
            c = i * _NW + wid
            rb = lax.rem(i, 2)

            @pl.when(c < n_blocks)
            def _():
                for r in range(2):
                    @pl.when(rb == r)
                    def _():
                        wait_load(r)

                        @pl.when(i >= 2)
                        def _():
                            wait_store(r)
                        transpose(r)
                        store_blk(c, r)

                        @pl.when((i + 2) * _NW + wid < n_blocks)
                        def _():
                            load_blk((i + 2) * _NW + wid, r)
            return carry

        load_blk(wid, 0)

        @pl.when(_NW + wid < n_blocks)
        def _():
            load_blk(_NW + wid, 1)

        lax.fori_loop(0, per_tile, body, 0)
        im = (n_blocks - 1 - wid) // _NW
        for p in range(2):
            @pl.when(lax.rem(im, 2) == p)
            def _():
                wait_store(p)
                wait_store(1 - p)

        ntail = V - V0

        @pl.when(wid == _NW - 1)
        def _():
            if ntail:
                pltpu.async_copy(wtail_hbm, at_v, sem_i[0])
                pltpu.make_async_copy(wtail_hbm, at_v, sem_i[0]).wait()

                def rows(rr, carry2):
                    cols = jax.lax.broadcast(rr, (16,))
                    for k in range(D // 16):
                        b_v[0][rr, pl.ds(16 * k, 16)] = plsc.load_gather(
                            at_v, [lanes + 16 * k, cols])
                    return carry2

                lax.fori_loop(0, ntail, rows, 0)
                pltpu.async_copy(b_v[0].at[pl.ds(0, ntail), :],
                                 out_hbm.at[pl.ds(V0, ntail), :], sem_o[0])
                pltpu.make_async_copy(
                    b_v[0].at[pl.ds(0, ntail), :],
                    out_hbm.at[pl.ds(0, ntail), :], sem_o[0]).wait()

    return widen_kernel


@functools.lru_cache(maxsize=None)
def _make_gather(B, D):
    b_per_w = B // _NW
    n_chunks = b_per_w // _GCHUNK
    assert n_chunks * _GCHUNK == b_per_w and n_chunks >= 4
    mesh = plsc.VectorSubcoreMesh(core_axis_name="c", subcore_axis_name="s")

    @functools.partial(
        pl.kernel,
        out_type=jax.ShapeDtypeStruct((B, D), jnp.float32),
        mesh=mesh,
        scratch_types=[
            pltpu.VMEM((b_per_w,), jnp.int32),
            pltpu.VMEM((_GCHUNK, 128), jnp.float32),
            pltpu.VMEM((_GCHUNK, 128), jnp.float32),
            pltpu.VMEM((_GCHUNK, D), jnp.float32),
            pltpu.SemaphoreType.DMA,
            pltpu.SemaphoreType.DMA,
            pltpu.SemaphoreType.DMA,
            pltpu.SemaphoreType.DMA,
        ],
        compiler_params=pltpu.CompilerParams(use_tc_tiling_on_sc=True),
    )
    def gather_kernel(idx_hbm, tab_hbm, out_hbm, idx_v, rows_v0, rows_v1,
                      obuf, sem_i, sg0, sg1, sem_s):
        rows_v = (rows_v0, rows_v1)
        sem_g = (sg0, sg1)
        wid = lax.axis_index("s") * _NC + lax.axis_index("c")
        base_w = wid * b_per_w

        def gather(g, b):
            pltpu.async_copy(
                tab_hbm.at[idx_v.at[pl.ds(g * _GCHUNK, _GCHUNK)]],
                rows_v[b], sem_g[b])

        def wait_gather(b):
            pltpu.make_async_copy(
                tab_hbm.at[idx_v.at[pl.ds(0, _GCHUNK)]],
                rows_v[b], sem_g[b]).wait()

        def compact(b):
            def rows(r8, carry2):
                for r in range(8):
                    for k in range(D // 16):
                        obuf[r8 * 8 + r, pl.ds(16 * k, 16)] = (
                            rows_v[b][r8 * 8 + r, pl.ds(16 * k, 16)])
                return carry2
            lax.fori_loop(0, _GCHUNK // 8, rows, 0)

        def store(g):
            pltpu.async_copy(
                obuf,
                out_hbm.at[pl.ds(base_w + g * _GCHUNK, _GCHUNK), :],
                sem_s)

        def wait_store():
            pltpu.make_async_copy(
                obuf, out_hbm.at[pl.ds(0, _GCHUNK), :], sem_s).wait()

        pltpu.async_copy(idx_hbm.at[pl.ds(base_w, b_per_w)], idx_v, sem_i)
        pltpu.make_async_copy(idx_hbm.at[pl.ds(0, b_per_w)],
                              idx_v, sem_i).wait()
        gather(0, 0)

        def body(g, carry):
            b = lax.rem(g, 2)

            @pl.when(b == 0)
            def _():
                wait_gather(0)

                @pl.when(g + 1 < n_chunks)
                def _():
                    gather(g + 1, 1)

                @pl.when(g >= 1)
                def _():
                    wait_store()
                compact(0)
                store(g)

            @pl.when(b == 1)
            def _():
                wait_gather(1)

                @pl.when(g + 1 < n_chunks)
                def _():
                    gather(g + 1, 0)
                wait_store()
                compact(1)
                store(g)
            return carry

        lax.fori_loop(0, n_chunks, body, 0)
        wait_store()

    return gather_kernel


def kernel(x, weight):
    shape = x.shape
    B = x.size
    V, D = weight.shape
    flat_idx = jnp.reshape(x.astype(jnp.int32), (B,))
    V0 = (V // _TBLK) * _TBLK
    wt = jnp.transpose(weight)
    wtail = jnp.transpose(weight[V0:, :])
    tab = _make_widen(V, D)(wt, wtail)
    out = _make_gather(B, D)(flat_idx, tab)
    return jnp.reshape(out, shape + (D,))


# speedup vs baseline: 1.9503x; 1.9503x over previous
import functools

import jax
import jax.numpy as jnp
from jax import lax
from jax.experimental import pallas as pl
from jax.experimental.pallas import tpu as pltpu
from jax.experimental.pallas import tpu_sc as plsc

_NC = 2
_NS = 16
_NW = _NC * _NS

_GCHUNK = 256


@functools.lru_cache(maxsize=None)
def _make_gather(B, D):
    b_per_w = B // _NW
    n_chunks = b_per_w // _GCHUNK
    assert n_chunks * _GCHUNK == b_per_w and n_chunks >= 4
    mesh = plsc.VectorSubcoreMesh(core_axis_name="c", subcore_axis_name="s")

    @functools.partial(
        pl.kernel,
        out_type=jax.ShapeDtypeStruct((B, D), jnp.float32),
        mesh=mesh,
        scratch_types=[
            pltpu.VMEM((b_per_w,), jnp.int32),
            pltpu.VMEM((_GCHUNK, 128), jnp.float32),
            pltpu.VMEM((_GCHUNK, 128), jnp.float32),
            pltpu.VMEM((_GCHUNK, D), jnp.float32),
            pltpu.SemaphoreType.DMA,
            pltpu.SemaphoreType.DMA,
            pltpu.SemaphoreType.DMA,
            pltpu.SemaphoreType.DMA,
        ],
        compiler_params=pltpu.CompilerParams(
            use_tc_tiling_on_sc=True, needs_layout_passes=False),
    )
    def gather_kernel(idx_hbm, tab_hbm, out_hbm, idx_v, rows_v0, rows_v1,
                      obuf, sem_i, sg0, sg1, sem_s):
        rows_v = (rows_v0, rows_v1)
        sem_g = (sg0, sg1)
        wid = lax.axis_index("s") * _NC + lax.axis_index("c")
        base_w = wid * b_per_w

        def gather(g, b):
            pltpu.async_copy(
                tab_hbm.at[idx_v.at[pl.ds(g * _GCHUNK, _GCHUNK)]],
                rows_v[b], sem_g[b])

        def wait_gather(b):
            pltpu.make_async_copy(
                tab_hbm.at[idx_v.at[pl.ds(0, _GCHUNK)]],
                rows_v[b], sem_g[b]).wait()

        def compact(b):
            def rows(r8, carry2):
                for r in range(8):
                    for k in range(D // 16):
                        obuf[r8 * 8 + r, pl.ds(16 * k, 16)] = (
                            rows_v[b][r8 * 8 + r, pl.ds(16 * k, 16)])
                return carry2
            lax.fori_loop(0, _GCHUNK // 8, rows, 0)

        def store(g):
            pltpu.async_copy(
                obuf,
                out_hbm.at[pl.ds(base_w + g * _GCHUNK, _GCHUNK), :],
                sem_s)

        def wait_store():
            pltpu.make_async_copy(
                obuf, out_hbm.at[pl.ds(0, _GCHUNK), :], sem_s).wait()

        pltpu.async_copy(idx_hbm.at[pl.ds(base_w, b_per_w)], idx_v, sem_i)
        pltpu.make_async_copy(idx_hbm.at[pl.ds(0, b_per_w)],
                              idx_v, sem_i).wait()
        gather(0, 0)

        def body(g, carry):
            b = lax.rem(g, 2)

            @pl.when(b == 0)
            def _():
                wait_gather(0)

                @pl.when(g + 1 < n_chunks)
                def _():
                    gather(g + 1, 1)

                @pl.when(g >= 1)
                def _():
                    wait_store()
                compact(0)
                store(g)

            @pl.when(b == 1)
            def _():
                wait_gather(1)

                @pl.when(g + 1 < n_chunks)
                def _():
                    gather(g + 1, 0)
                wait_store()
                compact(1)
                store(g)
            return carry

        lax.fori_loop(0, n_chunks, body, 0)
        wait_store()

    return gather_kernel


def kernel(x, weight):
    shape = x.shape
    B = x.size
    V, D = weight.shape
    flat_idx = jnp.reshape(x.astype(jnp.int32), (B,))
    tab = jnp.pad(weight, ((0, 0), (0, 128 - D)))
    out = _make_gather(B, D)(flat_idx, tab)
    return jnp.reshape(out, shape + (D,))


# speedup vs baseline: 1.9535x; 1.0017x over previous
import functools

import jax
import jax.numpy as jnp
from jax import lax
from jax.experimental import pallas as pl
from jax.experimental.pallas import tpu as pltpu
from jax.experimental.pallas import tpu_sc as plsc

_NC = 2
_NS = 16
_NW = _NC * _NS

_GCHUNK = 256


@functools.lru_cache(maxsize=None)
def _make_gather(B, D):
    b_per_w = B // _NW
    n_chunks = b_per_w // _GCHUNK
    assert n_chunks * _GCHUNK == b_per_w and n_chunks >= 4
    mesh = plsc.VectorSubcoreMesh(core_axis_name="c", subcore_axis_name="s")

    @functools.partial(
        pl.kernel,
        out_type=jax.ShapeDtypeStruct((B, 128), jnp.float32),
        mesh=mesh,
        scratch_types=[
            pltpu.VMEM((b_per_w,), jnp.int32),
            pltpu.VMEM((_GCHUNK, 128), jnp.float32),
            pltpu.VMEM((_GCHUNK, 128), jnp.float32),
            pltpu.SemaphoreType.DMA,
            pltpu.SemaphoreType.DMA,
            pltpu.SemaphoreType.DMA,
            pltpu.SemaphoreType.DMA,
        ],
        compiler_params=pltpu.CompilerParams(
            use_tc_tiling_on_sc=True, needs_layout_passes=False),
    )
    def gather_kernel(idx_hbm, tab_hbm, out_hbm, idx_v, rows_v0, rows_v1,
                      sem_i, sg0, sg1, sem_s):
        rows_v = (rows_v0, rows_v1)
        sem_g = (sg0, sg1)
        wid = lax.axis_index("s") * _NC + lax.axis_index("c")
        base_w = wid * b_per_w

        def gather(g, b):
            pltpu.async_copy(
                tab_hbm.at[idx_v.at[pl.ds(g * _GCHUNK, _GCHUNK)]],
                rows_v[b], sem_g[b])

        def wait_gather(b):
            pltpu.make_async_copy(
                tab_hbm.at[idx_v.at[pl.ds(0, _GCHUNK)]],
                rows_v[b], sem_g[b]).wait()

        def store(g, b):
            pltpu.async_copy(
                rows_v[b],
                out_hbm.at[pl.ds(base_w + g * _GCHUNK, _GCHUNK), :],
                sem_s)

        def wait_store(b):
            pltpu.make_async_copy(
                rows_v[b], out_hbm.at[pl.ds(0, _GCHUNK), :], sem_s).wait()

        pltpu.async_copy(idx_hbm.at[pl.ds(base_w, b_per_w)], idx_v, sem_i)
        pltpu.make_async_copy(idx_hbm.at[pl.ds(0, b_per_w)],
                              idx_v, sem_i).wait()
        gather(0, 0)

        def body(g, carry):
            b = lax.rem(g, 2)

            @pl.when(b == 0)
            def _():
                wait_gather(0)
                store(g, 0)
                wait_store(0)

                @pl.when(g + 2 < n_chunks)
                def _():
                    gather(g + 2, 0)

            @pl.when(b == 1)
            def _():
                wait_gather(1)
                store(g, 1)
                wait_store(1)

                @pl.when(g + 2 < n_chunks)
                def _():
                    gather(g + 2, 1)
            return carry

        gather(1, 1)
        lax.fori_loop(0, n_chunks, body, 0)

    return gather_kernel


def kernel(x, weight):
    shape = x.shape
    B = x.size
    V, D = weight.shape
    flat_idx = jnp.reshape(x.astype(jnp.int32), (B,))
    tab = jnp.pad(weight, ((0, 0), (0, 128 - D)))
    out = _make_gather(B, D)(flat_idx, tab)
    return jnp.reshape(out[:, :D], shape + (D,))


# speedup vs baseline: 1.9552x; 1.0008x over previous
import functools

import jax
import jax.numpy as jnp
from jax import lax
from jax.experimental import pallas as pl
from jax.experimental.pallas import tpu as pltpu
from jax.experimental.pallas import tpu_sc as plsc

_NC = 2
_NS = 16
_NW = _NC * _NS

_GCHUNK = 256


@functools.lru_cache(maxsize=None)
def _make_gather(B, D):
    b_per_w = B // _NW
    n_chunks = b_per_w // _GCHUNK
    assert n_chunks * _GCHUNK == b_per_w and n_chunks >= 4
    mesh = plsc.VectorSubcoreMesh(core_axis_name="c", subcore_axis_name="s")

    @functools.partial(
        pl.kernel,
        out_type=jax.ShapeDtypeStruct((B, 128), jnp.float32),
        mesh=mesh,
        scratch_types=[
            pltpu.VMEM((_GCHUNK,), jnp.int32),
            pltpu.VMEM((_GCHUNK,), jnp.int32),
            pltpu.VMEM((_GCHUNK,), jnp.int32),
            pltpu.VMEM((_GCHUNK, 128), jnp.float32),
            pltpu.VMEM((_GCHUNK, 128), jnp.float32),
            pltpu.VMEM((_GCHUNK, 128), jnp.float32),
            pltpu.SemaphoreType.DMA,
            pltpu.SemaphoreType.DMA,
            pltpu.SemaphoreType.DMA,
            pltpu.SemaphoreType.DMA,
            pltpu.SemaphoreType.DMA,
            pltpu.SemaphoreType.DMA,
            pltpu.SemaphoreType.DMA,
        ],
        compiler_params=pltpu.CompilerParams(
            use_tc_tiling_on_sc=True, needs_layout_passes=False),
    )
    def gather_kernel(idx_hbm, tab_hbm, out_hbm, idx_v0, idx_v1, idx_v2,
                      rows_v0, rows_v1, rows_v2,
                      si0, si1, si2, sg0, sg1, sg2, sem_s):
        idx_v = (idx_v0, idx_v1, idx_v2)
        sem_i = (si0, si1, si2)
        rows_v = (rows_v0, rows_v1, rows_v2)
        sem_g = (sg0, sg1, sg2)
        wid = lax.axis_index("s") * _NC + lax.axis_index("c")
        base_w = wid * b_per_w

        def load_idx(g, b):
            pltpu.async_copy(
                idx_hbm.at[pl.ds(base_w + g * _GCHUNK, _GCHUNK)],
                idx_v[b], sem_i[b])

        def wait_idx(b):
            pltpu.make_async_copy(
                idx_hbm.at[pl.ds(0, _GCHUNK)], idx_v[b], sem_i[b]).wait()

        def gather(b):
            pltpu.async_copy(tab_hbm.at[idx_v[b]], rows_v[b], sem_g[b])

        def wait_gather(b):
            pltpu.make_async_copy(
                tab_hbm.at[idx_v[b]], rows_v[b], sem_g[b]).wait()

        def store(g, b):
            pltpu.async_copy(
                rows_v[b],
                out_hbm.at[pl.ds(base_w + g * _GCHUNK, _GCHUNK), :],
                sem_s)

        def wait_store(b):
            pltpu.make_async_copy(
                rows_v[b], out_hbm.at[pl.ds(0, _GCHUNK), :], sem_s).wait()

        load_idx(0, 0)
        load_idx(1, 1)
        load_idx(2, 2)
        wait_idx(0)
        gather(0)

        def body(g, carry):
            b = lax.rem(g, 3)

            for r in range(3):
                @pl.when(b == r)
                def _():
                    wait_gather(r)
                    store(g, r)
                    wait_store(r)

                    @pl.when(g + 3 < n_chunks)
                    def _():
                        load_idx(g + 3, r)
                        wait_idx(r)
                        gather(r)
            return carry

        wait_idx(1)
        gather(1)
        wait_idx(2)
        gather(2)
        lax.fori_loop(0, n_chunks, body, 0)

    return gather_kernel


def kernel(x, weight):
    shape = x.shape
    B = x.size
    V, D = weight.shape
    flat_idx = jnp.reshape(x.astype(jnp.int32), (B,))
    tab = jnp.pad(weight, ((0, 0), (0, 128 - D)))
    out = _make_gather(B, D)(flat_idx, tab)
    return jnp.reshape(out[:, :D], shape + (D,))
